# Initial kernel scaffold; baseline (speedup 1.0000x reference)
import jax
import jax.numpy as jnp
from jax.experimental import pallas as pl


def kernel(x, edge_index, W0, b0, W1, b1):
    raise NotImplementedError("write your pallas kernel here")


# speedup vs baseline: 15.9248x; 15.9248x over previous
import functools

import jax
import jax.numpy as jnp
from jax import lax
from jax.experimental import pallas as pl
from jax.experimental.pallas import tpu as pltpu
from jax.experimental.pallas import tpu_sc as plsc

N = 10000
E = 320000
F = 128
HID = 256
C = 16
NC, NS = 2, 16
NW = NC * NS
NPAD = 10240
ROWS_PER_TILE = NPAD // NS
K = 128
NCHUNKS = E // K
FULL_ROUNDS = NCHUNKS // NW
EXTRA = NCHUNKS - FULL_ROUNDS * NW
R = 2000

_mesh = plsc.VectorSubcoreMesh(core_axis_name="c", subcore_axis_name="s")


def _make_sc_segsum(gather):
    @functools.partial(
        pl.kernel,
        out_type=jax.ShapeDtypeStruct((NC * NPAD, F), jnp.float32),
        mesh=_mesh,
        scratch_types=[
            pltpu.VMEM((K,), jnp.int32),
            pltpu.VMEM((K,), jnp.int32),
            pltpu.VMEM((K, F), jnp.float32),
            pltpu.VMEM_SHARED((NPAD, F), jnp.float32),
            pltpu.SemaphoreType.DMA,
        ],
    )
    def k(table, src, dst, zeros, out, src_v, dst_v, rows_v, acc, sem):
        cid = lax.axis_index("c")
        sid = lax.axis_index("s")
        wid = cid * NS + sid
        row0 = sid * ROWS_PER_TILE
        sl = pl.ds(row0, ROWS_PER_TILE)
        pltpu.sync_copy(zeros.at[sl], acc.at[sl])
        if not gather:
            pltpu.sync_copy(table, rows_v)
        plsc.subcore_barrier()

        n_chunks = FULL_ROUNDS + jnp.where(wid < EXTRA, 1, 0)

        def body(j, carry):
            base = pl.multiple_of((wid + j * NW) * K, K)
            pltpu.sync_copy(dst.at[pl.ds(base, K)], dst_v)
            if gather:
                pltpu.sync_copy(src.at[pl.ds(base, K)], src_v)
                pltpu.async_copy(table.at[src_v], rows_v, sem).wait()
            pltpu.sync_copy(rows_v, acc.at[dst_v], add=True)
            return carry

        lax.fori_loop(0, n_chunks, body, 0)
        plsc.subcore_barrier()
        out_row = cid * NPAD + row0
        pltpu.sync_copy(acc.at[sl], out.at[pl.ds(out_row, ROWS_PER_TILE)])

    return k


_sc_seg = _make_sc_segsum(gather=True)
_sc_degree = _make_sc_segsum(gather=False)


def _tc_scale_kernel(d0_ref, d1_ref, x_ref, xs_ref, dinv_ref):
    deg = d0_ref[...][:, :C] + d1_ref[...][:, :C] + 1.0
    dinv = lax.rsqrt(deg)
    dinv_ref[...] = dinv
    xs_ref[...] = x_ref[...] * dinv[:, :1]


def _tc_scale(p0, p1, x):
    return pl.pallas_call(
        _tc_scale_kernel,
        grid=(N // R,),
        in_specs=[
            pl.BlockSpec((R, F), lambda i: (i, 0)),
            pl.BlockSpec((R, F), lambda i: (i, 0)),
            pl.BlockSpec((R, F), lambda i: (i, 0)),
        ],
        out_specs=[
            pl.BlockSpec((R, F), lambda i: (i, 0)),
            pl.BlockSpec((R, C), lambda i: (i, 0)),
        ],
        out_shape=[
            jax.ShapeDtypeStruct((N, F), jnp.float32),
            jax.ShapeDtypeStruct((N, C), jnp.float32),
        ],
    )(p0, p1, x)


def _tc_dense_kernel(t0_ref, t1_ref, xs_ref, dinv_ref, W0_ref, b0_ref, W1_ref,
                     zs_ref):
    dinv = dinv_ref[...]
    g = (t0_ref[...] + t1_ref[...] + xs_ref[...]) * dinv[:, :1]
    h = jnp.dot(g, W0_ref[...], preferred_element_type=jnp.float32)
    h = jnp.maximum(h + b0_ref[...], 0.0)
    z = jnp.dot(h, W1_ref[...], preferred_element_type=jnp.float32)
    zs_ref[...] = jnp.concatenate(
        [z * dinv, jnp.zeros((z.shape[0], F - C), jnp.float32)], axis=1)


def _tc_dense(t0, t1, xs, dinv, W0, b0, W1):
    return pl.pallas_call(
        _tc_dense_kernel,
        grid=(N // R,),
        in_specs=[
            pl.BlockSpec((R, F), lambda i: (i, 0)),
            pl.BlockSpec((R, F), lambda i: (i, 0)),
            pl.BlockSpec((R, F), lambda i: (i, 0)),
            pl.BlockSpec((R, C), lambda i: (i, 0)),
            pl.BlockSpec((F, HID), lambda i: (0, 0)),
            pl.BlockSpec((1, HID), lambda i: (0, 0)),
            pl.BlockSpec((HID, C), lambda i: (0, 0)),
        ],
        out_specs=pl.BlockSpec((R, F), lambda i: (i, 0)),
        out_shape=jax.ShapeDtypeStruct((N, F), jnp.float32),
    )(t0, t1, xs, dinv, W0, b0, W1)


def _tc_final_kernel(u0_ref, u1_ref, zs_ref, dinv_ref, b1_ref, out_ref):
    u = (u0_ref[...][:, :C] + u1_ref[...][:, :C] + zs_ref[...][:, :C])
    out_ref[...] = u * dinv_ref[...] + b1_ref[...]


def _tc_final(u0, u1, zs, dinv, b1):
    return pl.pallas_call(
        _tc_final_kernel,
        grid=(N // R,),
        in_specs=[
            pl.BlockSpec((R, F), lambda i: (i, 0)),
            pl.BlockSpec((R, F), lambda i: (i, 0)),
            pl.BlockSpec((R, F), lambda i: (i, 0)),
            pl.BlockSpec((R, C), lambda i: (i, 0)),
            pl.BlockSpec((1, C), lambda i: (0, 0)),
        ],
        out_specs=pl.BlockSpec((R, C), lambda i: (i, 0)),
        out_shape=jax.ShapeDtypeStruct((N, C), jnp.float32),
    )(u0, u1, zs, dinv, b1)


def kernel(x, edge_index, W0, b0, W1, b1):
    src = edge_index[0]
    dst = edge_index[1]
    zerosF = jnp.zeros((NPAD, F), jnp.float32)
    onesK = jnp.ones((K, F), jnp.float32)

    dp = _sc_degree(onesK, src, dst, zerosF)
    xs, dinv = _tc_scale(dp[:N], dp[NPAD:NPAD + N], x)
    tp = _sc_seg(xs, src, dst, zerosF)
    zs = _tc_dense(tp[:N], tp[NPAD:NPAD + N], xs, dinv, W0,
                   b0.reshape(1, HID), W1)
    zs_pad = jnp.pad(zs, ((0, NPAD - N), (0, 0)))
    up = _sc_seg(zs_pad, src, dst, zerosF)
    out = _tc_final(up[:N], up[NPAD:NPAD + N], zs, dinv, b1.reshape(1, C))
    return out
